# Initial kernel scaffold; baseline (speedup 1.0000x reference)
import jax
import jax.numpy as jnp
from jax.experimental import pallas as pl


def kernel(x, edge_index, edge_attr, batch, W1, b1, W2, b2, Wf1, bf1, Wf2, bf2):
    raise NotImplementedError("write your pallas kernel here")


# speedup vs baseline: 26.0616x; 26.0616x over previous
import functools

import jax
import jax.numpy as jnp
from jax import lax
from jax.experimental import pallas as pl
from jax.experimental.pallas import tpu as pltpu
from jax.experimental.pallas import tpu_sc as plsc

NC = 2
NS = 16
LANES = 16
NW = NC * NS
CHUNK = 128


def _leaky(v):
    return jnp.where(v >= 0, v, 0.01 * v)


def _sc_mesh():
    return plsc.VectorSubcoreMesh(core_axis_name="c", subcore_axis_name="s")


def _deg_call(col3, ew3, zn, npad, nch):
    rpt = npad // NS

    @functools.partial(
        pl.kernel,
        out_type=jax.ShapeDtypeStruct((NC, npad), jnp.float32),
        mesh=_sc_mesh(),
        scratch_types=[
            pltpu.VMEM((nch, CHUNK), jnp.int32),
            pltpu.VMEM((nch, CHUNK), jnp.float32),
            pltpu.VMEM((rpt,), jnp.float32),
            pltpu.VMEM_SHARED((npad,), jnp.float32),
        ],
    )
    def deg_k(col3_hbm, ew3_hbm, zn_hbm, degp_hbm, colv, ewv, zv, acc):
        cid = lax.axis_index("c")
        sid = lax.axis_index("s")
        wid = sid * NC + cid
        pltpu.sync_copy(col3_hbm.at[wid], colv)
        pltpu.sync_copy(ew3_hbm.at[wid], ewv)
        pltpu.sync_copy(zn_hbm, zv)
        pltpu.sync_copy(zv, acc.at[pl.ds(sid * rpt, rpt)])
        plsc.subcore_barrier()

        def chunk(j, carry):
            pltpu.sync_copy(ewv.at[j], acc.at[colv.at[j]], add=True)
            return carry

        lax.fori_loop(0, nch, chunk, 0)
        plsc.subcore_barrier()
        pltpu.sync_copy(acc.at[pl.ds(sid * rpt, rpt)], zv)
        pltpu.sync_copy(zv, degp_hbm.at[cid, pl.ds(sid * rpt, rpt)])

    return deg_k(col3, ew3, zn)


def _prop1_call(row3, col3, ew3, xls, zrow, npad, nch, h, hreal):
    rpt = npad // NS

    @functools.partial(
        pl.kernel,
        out_type=jax.ShapeDtypeStruct((NC, npad, h), jnp.float32),
        mesh=_sc_mesh(),
        scratch_types=[
            pltpu.VMEM((nch, CHUNK), jnp.int32),
            pltpu.VMEM((nch, CHUNK), jnp.int32),
            pltpu.VMEM((nch, CHUNK), jnp.float32),
            pltpu.VMEM((CHUNK, h), jnp.float32),
            pltpu.SemaphoreType.DMA,
            pltpu.VMEM_SHARED((npad, h), jnp.float32),
        ],
    )
    def prop1_k(row3_hbm, col3_hbm, ew3_hbm, xls_hbm, zrow_hbm, hacc_hbm,
                rowv, colv, ewv, buf, sem, acc):
        cid = lax.axis_index("c")
        sid = lax.axis_index("s")
        wid = sid * NC + cid
        pltpu.sync_copy(row3_hbm.at[wid], rowv)
        pltpu.sync_copy(col3_hbm.at[wid], colv)
        pltpu.sync_copy(ew3_hbm.at[wid], ewv)
        pltpu.sync_copy(zrow_hbm, buf)
        for q in range(rpt // CHUNK):
            pltpu.sync_copy(buf, acc.at[pl.ds(sid * rpt + q * CHUNK, CHUNK)])
        plsc.subcore_barrier()

        def chunk(j, carry):
            pltpu.async_copy(xls_hbm.at[rowv.at[j]], buf, sem).wait()

            def scale(t, c2):
                w16 = ewv[j, pl.ds(t * LANES, LANES)]
                for i in range(LANES):
                    e = t * LANES + i
                    s = w16[i]
                    for q in range(hreal // LANES):
                        sl = pl.ds(q * LANES, LANES)
                        buf[e, sl] = buf[e, sl] * s
                return c2

            lax.fori_loop(0, CHUNK // LANES, scale, 0)
            pltpu.sync_copy(buf, acc.at[colv.at[j]], add=True)
            return carry

        lax.fori_loop(0, nch, chunk, 0)
        plsc.subcore_barrier()
        for q in range(rpt // CHUNK):
            off = sid * rpt + q * CHUNK
            pltpu.sync_copy(acc.at[pl.ds(off, CHUNK)], buf)
            pltpu.sync_copy(buf, hacc_hbm.at[cid, pl.ds(off, CHUNK)])

    return prop1_k(row3, col3, ew3, xls, zrow)


def _prop2_call(row3, col3, ew3, x2s, zn, npad, nch):
    rpt = npad // NS

    @functools.partial(
        pl.kernel,
        out_type=jax.ShapeDtypeStruct((NC, npad), jnp.float32),
        mesh=_sc_mesh(),
        scratch_types=[
            pltpu.VMEM((nch, CHUNK), jnp.int32),
            pltpu.VMEM((nch, CHUNK), jnp.int32),
            pltpu.VMEM((nch, CHUNK), jnp.float32),
            pltpu.VMEM((CHUNK,), jnp.float32),
            pltpu.VMEM((rpt,), jnp.float32),
            pltpu.SemaphoreType.DMA,
            pltpu.VMEM_SHARED((npad,), jnp.float32),
        ],
    )
    def prop2_k(row3_hbm, col3_hbm, ew3_hbm, x2s_hbm, zn_hbm, aacc_hbm,
                rowv, colv, ewv, msgv, zv, sem, acc):
        cid = lax.axis_index("c")
        sid = lax.axis_index("s")
        wid = sid * NC + cid
        pltpu.sync_copy(row3_hbm.at[wid], rowv)
        pltpu.sync_copy(col3_hbm.at[wid], colv)
        pltpu.sync_copy(ew3_hbm.at[wid], ewv)
        pltpu.sync_copy(zn_hbm, zv)
        pltpu.sync_copy(zv, acc.at[pl.ds(sid * rpt, rpt)])
        plsc.subcore_barrier()

        def chunk(j, carry):
            pltpu.async_copy(x2s_hbm.at[rowv.at[j]], msgv, sem).wait()
            for t in range(CHUNK // LANES):
                sl = pl.ds(t * LANES, LANES)
                msgv[sl] = msgv[sl] * ewv[j, sl]
            pltpu.sync_copy(msgv, acc.at[colv.at[j]], add=True)
            return carry

        lax.fori_loop(0, nch, chunk, 0)
        plsc.subcore_barrier()
        pltpu.sync_copy(acc.at[pl.ds(sid * rpt, rpt)], zv)
        pltpu.sync_copy(zv, aacc_hbm.at[cid, pl.ds(sid * rpt, rpt)])

    return prop2_k(row3, col3, ew3, x2s, zn)


def _lin1_call(xp, w1, degp, npad, d, h):
    blk = 256
    grid = (npad // blk,)

    def body(x_ref, w1_ref, degp_ref, xls_ref, dinv_ref):
        deg = degp_ref[0, :] + degp_ref[1, :] + 1.0
        dinv = jnp.where(deg > 0, lax.rsqrt(deg), 0.0)
        xl = jnp.dot(x_ref[...], w1_ref[...], preferred_element_type=jnp.float32)
        xls_ref[...] = xl * dinv[:, None]
        dinv_ref[...] = dinv[None, :]

    return pl.pallas_call(
        body,
        grid=grid,
        in_specs=[
            pl.BlockSpec((blk, d), lambda i: (i, 0)),
            pl.BlockSpec((d, h), lambda i: (0, 0)),
            pl.BlockSpec((NC, blk), lambda i: (0, i)),
        ],
        out_specs=[
            pl.BlockSpec((blk, h), lambda i: (i, 0)),
            pl.BlockSpec((1, blk), lambda i: (0, i)),
        ],
        out_shape=[
            jax.ShapeDtypeStruct((npad, h), jnp.float32),
            jax.ShapeDtypeStruct((1, npad), jnp.float32),
        ],
    )(xp, w1, degp)


def _lin2_call(hacc, xls, dinv, b1r, w2r, npad, h):
    blk = 256
    grid = (npad // blk,)

    def body(accp_ref, xls_ref, dinv_ref, b1_ref, w2_ref, x2s_ref):
        ea = accp_ref[0] + accp_ref[1]
        dinv = dinv_ref[0, :]
        pre = dinv[:, None] * (ea + xls_ref[...]) + b1_ref[0, :][None, :]
        h1 = _leaky(pre)
        xl2 = jnp.sum(h1 * w2_ref[0, :][None, :], axis=1)
        x2s_ref[...] = (dinv * xl2)[None, :]

    return pl.pallas_call(
        body,
        grid=grid,
        in_specs=[
            pl.BlockSpec((NC, blk, h), lambda i: (0, i, 0)),
            pl.BlockSpec((blk, h), lambda i: (i, 0)),
            pl.BlockSpec((1, blk), lambda i: (0, i)),
            pl.BlockSpec((1, h), lambda i: (0, 0)),
            pl.BlockSpec((1, h), lambda i: (0, 0)),
        ],
        out_specs=pl.BlockSpec((1, blk), lambda i: (0, i)),
        out_shape=jax.ShapeDtypeStruct((1, npad), jnp.float32),
    )(hacc, xls, dinv, b1r, w2r)


def _head_call(a0, a1, x2, dv, b2r, wf1, bf1r, wf2, bf2r, n, d2, out):
    nch = a0.shape[0]
    ch = a0.shape[2]

    def body(a0_ref, a1_ref, x2_ref, dv_ref, b2_ref, wf1_ref, bf1_ref,
             wf2_ref, bf2_ref, out_ref, z1):
        i = pl.program_id(0)
        pre = dv_ref[0, 0, :] * (a0_ref[0, 0, :] + a1_ref[0, 0, :]
                                 + x2_ref[0, 0, :]) + b2_ref[0, 0]
        h2 = _leaky(pre)
        part = jnp.dot(h2[None, :], wf1_ref[...],
                       preferred_element_type=jnp.float32)

        @pl.when(i == 0)
        def _():
            z1[0:1, :] = part

        @pl.when(i > 0)
        def _():
            z1[0:1, :] = z1[0:1, :] + part

        @pl.when(i == pl.num_programs(0) - 1)
        def _():
            zz = z1[0:1, :] + bf1_ref[...]
            aa = _leaky(zz)
            z2 = jnp.dot(aa, wf2_ref[...],
                         preferred_element_type=jnp.float32) + bf2_ref[...]
            a2 = _leaky(z2)
            m = jnp.max(a2, axis=1, keepdims=True)
            ex = jnp.exp(a2 - m)
            out_ref[...] = ex / jnp.sum(ex, axis=1, keepdims=True)

    return pl.pallas_call(
        body,
        grid=(nch,),
        in_specs=[
            pl.BlockSpec((1, 1, ch), lambda i: (i, 0, 0)),
            pl.BlockSpec((1, 1, ch), lambda i: (i, 0, 0)),
            pl.BlockSpec((1, 1, ch), lambda i: (i, 0, 0)),
            pl.BlockSpec((1, 1, ch), lambda i: (i, 0, 0)),
            pl.BlockSpec((1, 1), lambda i: (0, 0)),
            pl.BlockSpec((ch, d2), lambda i: (i, 0)),
            pl.BlockSpec((1, d2), lambda i: (0, 0)),
            pl.BlockSpec(wf2.shape, lambda i: (0, 0)),
            pl.BlockSpec((1, out), lambda i: (0, 0)),
        ],
        out_specs=pl.BlockSpec((1, out), lambda i: (0, 0)),
        out_shape=jax.ShapeDtypeStruct((1, out), jnp.float32),
        scratch_shapes=[pltpu.VMEM((8, d2), jnp.float32)],
    )(a0, a1, x2, dv, b2r, wf1, bf1r, wf2, bf2r)


def kernel(x, edge_index, edge_attr, batch, W1, b1, W2, b2, Wf1, bf1, Wf2, bf2):
    n, d = x.shape
    h = W1.shape[1]
    d2 = Wf1.shape[1]
    out = Wf2.shape[1]
    edges = edge_index.shape[1]

    npad = -(-n // (NS * CHUNK)) * (NS * CHUNK)
    nch = -(-edges // (NW * CHUNK))
    epad = NW * nch * CHUNK - edges

    row = edge_index[0]
    col = edge_index[1]
    pidx = jnp.arange(epad, dtype=jnp.int32) % n
    rowp = jnp.concatenate([row, pidx]).reshape(NW, nch, CHUNK)
    colp = jnp.concatenate([col, pidx]).reshape(NW, nch, CHUNK)
    ewp = jnp.concatenate(
        [edge_attr, jnp.zeros((epad,), jnp.float32)]).reshape(NW, nch, CHUNK)

    hpad = 128
    W1p = jnp.pad(W1, ((0, 0), (0, hpad - h)))
    b1p = jnp.pad(b1, (0, hpad - h))
    W2p = jnp.pad(W2.reshape(-1), (0, hpad - h))

    zn = jnp.zeros((npad // NS,), jnp.float32)
    zrow = jnp.zeros((CHUNK, hpad), jnp.float32)
    xp = jnp.pad(x, ((0, npad - n), (0, 0)))

    degp = _deg_call(colp, ewp, zn, npad, nch)
    xls, dinv = _lin1_call(xp, W1p, degp, npad, d, hpad)

    hacc = _prop1_call(rowp, colp, ewp, xls, zrow, npad, nch, hpad, h)
    x2s = _lin2_call(hacc, xls, dinv, b1p.reshape(1, hpad),
                     W2p.reshape(1, hpad), npad, hpad)

    aacc = _prop2_call(rowp, colp, ewp, x2s.reshape(npad), zn, npad, nch)

    ch = 2000
    ng = n // ch
    a0 = aacc[0, :n].reshape(ng, 1, ch)
    a1 = aacc[1, :n].reshape(ng, 1, ch)
    x2r = x2s[0, :n].reshape(ng, 1, ch)
    dvr = dinv[0, :n].reshape(ng, 1, ch)
    return _head_call(a0, a1, x2r, dvr, b2.reshape(1, 1), Wf1,
                      bf1.reshape(1, d2), Wf2, bf2.reshape(1, out),
                      n, d2, out)


# speedup vs baseline: 27.4015x; 1.0514x over previous
import functools

import jax
import jax.numpy as jnp
from jax import lax
from jax.experimental import pallas as pl
from jax.experimental.pallas import tpu as pltpu
from jax.experimental.pallas import tpu_sc as plsc

NC = 2
NS = 16
LANES = 16
NW = NC * NS
CHUNK = 128


def _leaky(v):
    return jnp.where(v >= 0, v, 0.01 * v)


def _sc_mesh():
    return plsc.VectorSubcoreMesh(core_axis_name="c", subcore_axis_name="s")


def _deg_call(col3, ew3, zn, npad, nch):
    rpt = npad // NS

    @functools.partial(
        pl.kernel,
        out_type=jax.ShapeDtypeStruct((NC, npad), jnp.float32),
        mesh=_sc_mesh(),
        scratch_types=[
            pltpu.VMEM((nch, CHUNK), jnp.int32),
            pltpu.VMEM((nch, CHUNK), jnp.float32),
            pltpu.VMEM((rpt,), jnp.float32),
            pltpu.VMEM_SHARED((npad,), jnp.float32),
        ],
    )
    def deg_k(col3_hbm, ew3_hbm, zn_hbm, degp_hbm, colv, ewv, zv, acc):
        cid = lax.axis_index("c")
        sid = lax.axis_index("s")
        wid = sid * NC + cid
        pltpu.sync_copy(col3_hbm.at[wid], colv)
        pltpu.sync_copy(ew3_hbm.at[wid], ewv)
        pltpu.sync_copy(zn_hbm, zv)
        pltpu.sync_copy(zv, acc.at[pl.ds(sid * rpt, rpt)])
        plsc.subcore_barrier()

        def chunk(j, carry):
            pltpu.sync_copy(ewv.at[j], acc.at[colv.at[j]], add=True)
            return carry

        lax.fori_loop(0, nch, chunk, 0)
        plsc.subcore_barrier()
        pltpu.sync_copy(acc.at[pl.ds(sid * rpt, rpt)], zv)
        pltpu.sync_copy(zv, degp_hbm.at[cid, pl.ds(sid * rpt, rpt)])

    return deg_k(col3, ew3, zn)


def _prop1_call(row3, col3, ew3, xls, zrow, npad, nch, h, hreal):
    rpt = npad // NS
    SUB = CHUNK // 2

    @functools.partial(
        pl.kernel,
        out_type=jax.ShapeDtypeStruct((NC, npad, h), jnp.float32),
        mesh=_sc_mesh(),
        scratch_types=[
            pltpu.VMEM((nch, CHUNK), jnp.int32),
            pltpu.VMEM((nch, CHUNK), jnp.int32),
            pltpu.VMEM((nch, CHUNK), jnp.float32),
            pltpu.VMEM((SUB, h), jnp.float32),
            pltpu.VMEM((SUB, h), jnp.float32),
            pltpu.SemaphoreType.DMA,
            pltpu.SemaphoreType.DMA,
            pltpu.VMEM_SHARED((npad, h), jnp.float32),
        ],
    )
    def prop1_k(row3_hbm, col3_hbm, ew3_hbm, xls_hbm, zrow_hbm, hacc_hbm,
                rowv, colv, ewv, buf0, buf1, sem0, sem1, acc):
        cid = lax.axis_index("c")
        sid = lax.axis_index("s")
        wid = sid * NC + cid
        pltpu.sync_copy(row3_hbm.at[wid], rowv)
        pltpu.sync_copy(col3_hbm.at[wid], colv)
        pltpu.sync_copy(ew3_hbm.at[wid], ewv)
        for q in range(rpt // CHUNK):
            pltpu.sync_copy(
                zrow_hbm, acc.at[pl.ds(sid * rpt + q * CHUNK, CHUNK)])
        plsc.subcore_barrier()

        def scale_scatter(j, half, buf):
            def scale(t, c2):
                w16 = ewv[j, pl.ds(half * SUB + t * LANES, LANES)]
                for i in range(LANES):
                    e = t * LANES + i
                    s = w16[i]
                    for q in range(hreal // LANES):
                        sl = pl.ds(q * LANES, LANES)
                        buf[e, sl] = buf[e, sl] * s
                return c2

            lax.fori_loop(0, SUB // LANES, scale, 0)
            pltpu.sync_copy(
                buf, acc.at[colv.at[j, pl.ds(half * SUB, SUB)]], add=True)

        pltpu.async_copy(
            xls_hbm.at[rowv.at[0, pl.ds(0, SUB)]], buf0, sem0)

        def step(j, carry):
            pltpu.make_async_copy(
                xls_hbm.at[rowv.at[j, pl.ds(0, SUB)]], buf0, sem0).wait()
            pltpu.async_copy(
                xls_hbm.at[rowv.at[j, pl.ds(SUB, SUB)]], buf1, sem1)
            scale_scatter(j, 0, buf0)
            pltpu.make_async_copy(
                xls_hbm.at[rowv.at[j, pl.ds(SUB, SUB)]], buf1, sem1).wait()
            jn = jnp.where(j + 1 < nch, j + 1, 0)
            pltpu.async_copy(
                xls_hbm.at[rowv.at[jn, pl.ds(0, SUB)]], buf0, sem0)
            scale_scatter(j, 1, buf1)
            return carry

        lax.fori_loop(0, nch, step, 0)
        pltpu.make_async_copy(
            xls_hbm.at[rowv.at[0, pl.ds(0, SUB)]], buf0, sem0).wait()
        plsc.subcore_barrier()
        for q in range(rpt // SUB):
            off = sid * rpt + q * SUB
            pltpu.sync_copy(acc.at[pl.ds(off, SUB)], buf0)
            pltpu.sync_copy(buf0, hacc_hbm.at[cid, pl.ds(off, SUB)])

    return prop1_k(row3, col3, ew3, xls, zrow)


def _prop2_call(row3, col3, ew3, x2s, zn, npad, nch):
    rpt = npad // NS

    @functools.partial(
        pl.kernel,
        out_type=jax.ShapeDtypeStruct((NC, npad), jnp.float32),
        mesh=_sc_mesh(),
        scratch_types=[
            pltpu.VMEM((nch, CHUNK), jnp.int32),
            pltpu.VMEM((nch, CHUNK), jnp.int32),
            pltpu.VMEM((nch, CHUNK), jnp.float32),
            pltpu.VMEM((CHUNK,), jnp.float32),
            pltpu.VMEM((CHUNK,), jnp.float32),
            pltpu.VMEM((rpt,), jnp.float32),
            pltpu.SemaphoreType.DMA,
            pltpu.SemaphoreType.DMA,
            pltpu.VMEM_SHARED((npad,), jnp.float32),
        ],
    )
    def prop2_k(row3_hbm, col3_hbm, ew3_hbm, x2s_hbm, zn_hbm, aacc_hbm,
                rowv, colv, ewv, msg0, msg1, zv, sem0, sem1, acc):
        cid = lax.axis_index("c")
        sid = lax.axis_index("s")
        wid = sid * NC + cid
        pltpu.sync_copy(row3_hbm.at[wid], rowv)
        pltpu.sync_copy(col3_hbm.at[wid], colv)
        pltpu.sync_copy(ew3_hbm.at[wid], ewv)
        pltpu.sync_copy(zn_hbm, zv)
        pltpu.sync_copy(zv, acc.at[pl.ds(sid * rpt, rpt)])
        plsc.subcore_barrier()

        def scale_scatter(j, msgv):
            for t in range(CHUNK // LANES):
                sl = pl.ds(t * LANES, LANES)
                msgv[sl] = msgv[sl] * ewv[j, sl]
            pltpu.sync_copy(msgv, acc.at[colv.at[j]], add=True)

        pltpu.async_copy(x2s_hbm.at[rowv.at[0]], msg0, sem0)

        def pair(p, carry):
            j0 = 2 * p
            pltpu.make_async_copy(x2s_hbm.at[rowv.at[j0]], msg0, sem0).wait()
            pltpu.async_copy(x2s_hbm.at[rowv.at[j0 + 1]], msg1, sem1)
            scale_scatter(j0, msg0)
            pltpu.make_async_copy(
                x2s_hbm.at[rowv.at[j0 + 1]], msg1, sem1).wait()
            jn = jnp.where(j0 + 2 < nch, j0 + 2, 0)
            pltpu.async_copy(x2s_hbm.at[rowv.at[jn]], msg0, sem0)
            scale_scatter(j0 + 1, msg1)
            return carry

        lax.fori_loop(0, nch // 2, pair, 0)
        pltpu.make_async_copy(x2s_hbm.at[rowv.at[0]], msg0, sem0).wait()
        plsc.subcore_barrier()
        pltpu.sync_copy(acc.at[pl.ds(sid * rpt, rpt)], zv)
        pltpu.sync_copy(zv, aacc_hbm.at[cid, pl.ds(sid * rpt, rpt)])

    return prop2_k(row3, col3, ew3, x2s, zn)


def _lin1_call(xp, w1, degp, npad, d, h):
    blk = 256
    grid = (npad // blk,)

    def body(x_ref, w1_ref, degp_ref, xls_ref, dinv_ref):
        deg = degp_ref[0, :] + degp_ref[1, :] + 1.0
        dinv = jnp.where(deg > 0, lax.rsqrt(deg), 0.0)
        xl = jnp.dot(x_ref[...], w1_ref[...], preferred_element_type=jnp.float32)
        xls_ref[...] = xl * dinv[:, None]
        dinv_ref[...] = dinv[None, :]

    return pl.pallas_call(
        body,
        grid=grid,
        in_specs=[
            pl.BlockSpec((blk, d), lambda i: (i, 0)),
            pl.BlockSpec((d, h), lambda i: (0, 0)),
            pl.BlockSpec((NC, blk), lambda i: (0, i)),
        ],
        out_specs=[
            pl.BlockSpec((blk, h), lambda i: (i, 0)),
            pl.BlockSpec((1, blk), lambda i: (0, i)),
        ],
        out_shape=[
            jax.ShapeDtypeStruct((npad, h), jnp.float32),
            jax.ShapeDtypeStruct((1, npad), jnp.float32),
        ],
    )(xp, w1, degp)


def _lin2_call(hacc, xls, dinv, b1r, w2r, npad, h):
    blk = 256
    grid = (npad // blk,)

    def body(accp_ref, xls_ref, dinv_ref, b1_ref, w2_ref, x2s_ref):
        ea = accp_ref[0] + accp_ref[1]
        dinv = dinv_ref[0, :]
        pre = dinv[:, None] * (ea + xls_ref[...]) + b1_ref[0, :][None, :]
        h1 = _leaky(pre)
        xl2 = jnp.sum(h1 * w2_ref[0, :][None, :], axis=1)
        x2s_ref[...] = (dinv * xl2)[None, :]

    return pl.pallas_call(
        body,
        grid=grid,
        in_specs=[
            pl.BlockSpec((NC, blk, h), lambda i: (0, i, 0)),
            pl.BlockSpec((blk, h), lambda i: (i, 0)),
            pl.BlockSpec((1, blk), lambda i: (0, i)),
            pl.BlockSpec((1, h), lambda i: (0, 0)),
            pl.BlockSpec((1, h), lambda i: (0, 0)),
        ],
        out_specs=pl.BlockSpec((1, blk), lambda i: (0, i)),
        out_shape=jax.ShapeDtypeStruct((1, npad), jnp.float32),
    )(hacc, xls, dinv, b1r, w2r)


def _head_call(a0, a1, x2, dv, b2r, wf1, bf1r, wf2, bf2r, n, d2, out):
    nch = a0.shape[0]
    ch = a0.shape[2]

    def body(a0_ref, a1_ref, x2_ref, dv_ref, b2_ref, wf1_ref, bf1_ref,
             wf2_ref, bf2_ref, out_ref, z1):
        i = pl.program_id(0)
        pre = dv_ref[0, 0, :] * (a0_ref[0, 0, :] + a1_ref[0, 0, :]
                                 + x2_ref[0, 0, :]) + b2_ref[0, 0]
        h2 = _leaky(pre)
        part = jnp.dot(h2[None, :], wf1_ref[...],
                       preferred_element_type=jnp.float32)

        @pl.when(i == 0)
        def _():
            z1[0:1, :] = part

        @pl.when(i > 0)
        def _():
            z1[0:1, :] = z1[0:1, :] + part

        @pl.when(i == pl.num_programs(0) - 1)
        def _():
            zz = z1[0:1, :] + bf1_ref[...]
            aa = _leaky(zz)
            z2 = jnp.dot(aa, wf2_ref[...],
                         preferred_element_type=jnp.float32) + bf2_ref[...]
            a2 = _leaky(z2)
            m = jnp.max(a2, axis=1, keepdims=True)
            ex = jnp.exp(a2 - m)
            out_ref[...] = ex / jnp.sum(ex, axis=1, keepdims=True)

    return pl.pallas_call(
        body,
        grid=(nch,),
        in_specs=[
            pl.BlockSpec((1, 1, ch), lambda i: (i, 0, 0)),
            pl.BlockSpec((1, 1, ch), lambda i: (i, 0, 0)),
            pl.BlockSpec((1, 1, ch), lambda i: (i, 0, 0)),
            pl.BlockSpec((1, 1, ch), lambda i: (i, 0, 0)),
            pl.BlockSpec((1, 1), lambda i: (0, 0)),
            pl.BlockSpec((ch, d2), lambda i: (i, 0)),
            pl.BlockSpec((1, d2), lambda i: (0, 0)),
            pl.BlockSpec(wf2.shape, lambda i: (0, 0)),
            pl.BlockSpec((1, out), lambda i: (0, 0)),
        ],
        out_specs=pl.BlockSpec((1, out), lambda i: (0, 0)),
        out_shape=jax.ShapeDtypeStruct((1, out), jnp.float32),
        scratch_shapes=[pltpu.VMEM((8, d2), jnp.float32)],
    )(a0, a1, x2, dv, b2r, wf1, bf1r, wf2, bf2r)


def kernel(x, edge_index, edge_attr, batch, W1, b1, W2, b2, Wf1, bf1, Wf2, bf2):
    n, d = x.shape
    h = W1.shape[1]
    d2 = Wf1.shape[1]
    out = Wf2.shape[1]
    edges = edge_index.shape[1]

    npad = -(-n // (NS * CHUNK)) * (NS * CHUNK)
    nch = -(-edges // (NW * CHUNK))
    nch = nch + (nch % 2)
    epad = NW * nch * CHUNK - edges

    row = edge_index[0]
    col = edge_index[1]
    pidx = jnp.arange(epad, dtype=jnp.int32) % n
    rowp = jnp.concatenate([row, pidx]).reshape(NW, nch, CHUNK)
    colp = jnp.concatenate([col, pidx]).reshape(NW, nch, CHUNK)
    ewp = jnp.concatenate(
        [edge_attr, jnp.zeros((epad,), jnp.float32)]).reshape(NW, nch, CHUNK)

    hpad = 128
    W1p = jnp.pad(W1, ((0, 0), (0, hpad - h)))
    b1p = jnp.pad(b1, (0, hpad - h))
    W2p = jnp.pad(W2.reshape(-1), (0, hpad - h))

    zn = jnp.zeros((npad // NS,), jnp.float32)
    zrow = jnp.zeros((CHUNK, hpad), jnp.float32)
    xp = jnp.pad(x, ((0, npad - n), (0, 0)))

    degp = _deg_call(colp, ewp, zn, npad, nch)
    xls, dinv = _lin1_call(xp, W1p, degp, npad, d, hpad)

    hacc = _prop1_call(rowp, colp, ewp, xls, zrow, npad, nch, hpad, h)
    x2s = _lin2_call(hacc, xls, dinv, b1p.reshape(1, hpad),
                     W2p.reshape(1, hpad), npad, hpad)

    aacc = _prop2_call(rowp, colp, ewp, x2s.reshape(npad), zn, npad, nch)

    ch = 2000
    ng = n // ch
    a0 = aacc[0, :n].reshape(ng, 1, ch)
    a1 = aacc[1, :n].reshape(ng, 1, ch)
    x2r = x2s[0, :n].reshape(ng, 1, ch)
    dvr = dinv[0, :n].reshape(ng, 1, ch)
    return _head_call(a0, a1, x2r, dvr, b2.reshape(1, 1), Wf1,
                      bf1.reshape(1, d2), Wf2, bf2.reshape(1, out),
                      n, d2, out)


# speedup vs baseline: 33.1819x; 1.2109x over previous
import functools

import jax
import jax.numpy as jnp
from jax import lax
from jax.experimental import pallas as pl
from jax.experimental.pallas import tpu as pltpu
from jax.experimental.pallas import tpu_sc as plsc

NC = 2
NS = 16
LANES = 16
NW = NC * NS
CHUNK = 128


def _leaky(v):
    return jnp.where(v >= 0, v, 0.01 * v)


def _sc_mesh():
    return plsc.VectorSubcoreMesh(core_axis_name="c", subcore_axis_name="s")


def _deg_call(col3, ew3, zn, npad, nch):
    rpt = npad // NS

    @functools.partial(
        pl.kernel,
        out_type=jax.ShapeDtypeStruct((NC, npad), jnp.float32),
        mesh=_sc_mesh(),
        scratch_types=[
            pltpu.VMEM((nch, CHUNK), jnp.int32),
            pltpu.VMEM((nch, CHUNK), jnp.float32),
            pltpu.VMEM((rpt,), jnp.float32),
            pltpu.VMEM_SHARED((npad,), jnp.float32),
        ],
    )
    def deg_k(col3_hbm, ew3_hbm, zn_hbm, degp_hbm, colv, ewv, zv, acc):
        cid = lax.axis_index("c")
        sid = lax.axis_index("s")
        wid = sid * NC + cid
        pltpu.sync_copy(col3_hbm.at[wid], colv)
        pltpu.sync_copy(ew3_hbm.at[wid], ewv)
        pltpu.sync_copy(zn_hbm, zv)
        pltpu.sync_copy(zv, acc.at[pl.ds(sid * rpt, rpt)])
        plsc.subcore_barrier()

        def chunk(j, carry):
            pltpu.sync_copy(ewv.at[j], acc.at[colv.at[j]], add=True)
            return carry

        lax.fori_loop(0, nch, chunk, 0)
        plsc.subcore_barrier()
        pltpu.sync_copy(acc.at[pl.ds(sid * rpt, rpt)], zv)
        pltpu.sync_copy(zv, degp_hbm.at[cid, pl.ds(sid * rpt, rpt)])

    return deg_k(col3, ew3, zn)


def _prop1_call(row3, col3, ew3, xls, zrow, npad, nch, h, hreal):
    rpt = npad // NS
    SUB = CHUNK // 2

    @functools.partial(
        pl.kernel,
        out_type=jax.ShapeDtypeStruct((NC, npad, h), jnp.float32),
        mesh=_sc_mesh(),
        scratch_types=[
            pltpu.VMEM((nch, CHUNK), jnp.int32),
            pltpu.VMEM((nch, CHUNK), jnp.int32),
            pltpu.VMEM((nch, CHUNK), jnp.float32),
            pltpu.VMEM((SUB, h), jnp.float32),
            pltpu.VMEM((SUB, h), jnp.float32),
            pltpu.SemaphoreType.DMA,
            pltpu.SemaphoreType.DMA,
            pltpu.VMEM_SHARED((npad, h), jnp.float32),
        ],
    )
    def prop1_k(row3_hbm, col3_hbm, ew3_hbm, xls_hbm, zrow_hbm, hacc_hbm,
                rowv, colv, ewv, buf0, buf1, sem0, sem1, acc):
        cid = lax.axis_index("c")
        sid = lax.axis_index("s")
        wid = sid * NC + cid
        pltpu.sync_copy(row3_hbm.at[wid], rowv)
        pltpu.sync_copy(col3_hbm.at[wid], colv)
        pltpu.sync_copy(ew3_hbm.at[wid], ewv)
        for q in range(rpt // CHUNK):
            pltpu.sync_copy(
                zrow_hbm, acc.at[pl.ds(sid * rpt + q * CHUNK, CHUNK)])
        plsc.subcore_barrier()

        def scale_scatter(j, half, buf):
            def scale(t, c2):
                w16 = ewv[j, pl.ds(half * SUB + t * LANES, LANES)]
                for i in range(LANES):
                    e = t * LANES + i
                    s = w16[i]
                    for q in range(hreal // LANES):
                        sl = pl.ds(q * LANES, LANES)
                        buf[e, sl] = buf[e, sl] * s
                return c2

            lax.fori_loop(0, SUB // LANES, scale, 0)
            pltpu.sync_copy(
                buf, acc.at[colv.at[j, pl.ds(half * SUB, SUB)]], add=True)

        pltpu.async_copy(
            xls_hbm.at[rowv.at[0, pl.ds(0, SUB)]], buf0, sem0)

        def step(j, carry):
            pltpu.make_async_copy(
                xls_hbm.at[rowv.at[j, pl.ds(0, SUB)]], buf0, sem0).wait()
            pltpu.async_copy(
                xls_hbm.at[rowv.at[j, pl.ds(SUB, SUB)]], buf1, sem1)
            scale_scatter(j, 0, buf0)
            pltpu.make_async_copy(
                xls_hbm.at[rowv.at[j, pl.ds(SUB, SUB)]], buf1, sem1).wait()
            jn = jnp.where(j + 1 < nch, j + 1, 0)
            pltpu.async_copy(
                xls_hbm.at[rowv.at[jn, pl.ds(0, SUB)]], buf0, sem0)
            scale_scatter(j, 1, buf1)
            return carry

        lax.fori_loop(0, nch, step, 0)
        pltpu.make_async_copy(
            xls_hbm.at[rowv.at[0, pl.ds(0, SUB)]], buf0, sem0).wait()
        plsc.subcore_barrier()
        for q in range(rpt // SUB):
            off = sid * rpt + q * SUB
            pltpu.sync_copy(acc.at[pl.ds(off, SUB)], buf0)
            pltpu.sync_copy(buf0, hacc_hbm.at[cid, pl.ds(off, SUB)])

    return prop1_k(row3, col3, ew3, xls, zrow)


def _prop2_call(row3, col3, ew3, x2s, npad, nch):

    @functools.partial(
        pl.kernel,
        out_type=jax.ShapeDtypeStruct((NW, npad), jnp.float32),
        mesh=_sc_mesh(),
        compiler_params=pltpu.CompilerParams(needs_layout_passes=False),
        scratch_types=[
            pltpu.VMEM((nch, CHUNK), jnp.int32),
            pltpu.VMEM((nch, CHUNK), jnp.int32),
            pltpu.VMEM((nch, CHUNK), jnp.float32),
            pltpu.VMEM((npad,), jnp.float32),
            pltpu.VMEM((npad,), jnp.float32),
        ],
    )
    def prop2_k(row3_hbm, col3_hbm, ew3_hbm, x2s_hbm, aacc_hbm,
                rowv, colv, ewv, x2v, acc):
        cid = lax.axis_index("c")
        sid = lax.axis_index("s")
        wid = sid * NC + cid
        pltpu.sync_copy(row3_hbm.at[wid], rowv)
        pltpu.sync_copy(col3_hbm.at[wid], colv)
        pltpu.sync_copy(ew3_hbm.at[wid], ewv)
        pltpu.sync_copy(x2s_hbm, x2v)

        zv16 = jnp.zeros((LANES,), jnp.float32)

        def zero(i, carry):
            acc[pl.ds(i * LANES, LANES)] = zv16
            return carry

        lax.fori_loop(0, npad // LANES, zero, 0)

        def chunk(j, carry):
            for t in range(CHUNK // LANES):
                sl = pl.ds(t * LANES, LANES)
                vals = plsc.load_gather(x2v, [rowv[j, sl]])
                plsc.addupdate_scatter(
                    acc, [colv[j, sl]], vals * ewv[j, sl])
            return carry

        lax.fori_loop(0, nch, chunk, 0)
        pltpu.sync_copy(acc, aacc_hbm.at[wid])

    return prop2_k(row3, col3, ew3, x2s)


def _lin1_call(xp, w1, degp, npad, d, h):
    blk = 256
    grid = (npad // blk,)

    def body(x_ref, w1_ref, degp_ref, xls_ref, dinv_ref):
        deg = degp_ref[0, :] + degp_ref[1, :] + 1.0
        dinv = jnp.where(deg > 0, lax.rsqrt(deg), 0.0)
        xl = jnp.dot(x_ref[...], w1_ref[...], preferred_element_type=jnp.float32)
        xls_ref[...] = xl * dinv[:, None]
        dinv_ref[...] = dinv[None, :]

    return pl.pallas_call(
        body,
        grid=grid,
        in_specs=[
            pl.BlockSpec((blk, d), lambda i: (i, 0)),
            pl.BlockSpec((d, h), lambda i: (0, 0)),
            pl.BlockSpec((NC, blk), lambda i: (0, i)),
        ],
        out_specs=[
            pl.BlockSpec((blk, h), lambda i: (i, 0)),
            pl.BlockSpec((1, blk), lambda i: (0, i)),
        ],
        out_shape=[
            jax.ShapeDtypeStruct((npad, h), jnp.float32),
            jax.ShapeDtypeStruct((1, npad), jnp.float32),
        ],
    )(xp, w1, degp)


def _lin2_call(hacc, xls, dinv, b1r, w2r, npad, h):
    blk = 256
    grid = (npad // blk,)

    def body(accp_ref, xls_ref, dinv_ref, b1_ref, w2_ref, x2s_ref):
        ea = accp_ref[0] + accp_ref[1]
        dinv = dinv_ref[0, :]
        pre = dinv[:, None] * (ea + xls_ref[...]) + b1_ref[0, :][None, :]
        h1 = _leaky(pre)
        xl2 = jnp.sum(h1 * w2_ref[0, :][None, :], axis=1)
        x2s_ref[...] = (dinv * xl2)[None, :]

    return pl.pallas_call(
        body,
        grid=grid,
        in_specs=[
            pl.BlockSpec((NC, blk, h), lambda i: (0, i, 0)),
            pl.BlockSpec((blk, h), lambda i: (i, 0)),
            pl.BlockSpec((1, blk), lambda i: (0, i)),
            pl.BlockSpec((1, h), lambda i: (0, 0)),
            pl.BlockSpec((1, h), lambda i: (0, 0)),
        ],
        out_specs=pl.BlockSpec((1, blk), lambda i: (0, i)),
        out_shape=jax.ShapeDtypeStruct((1, npad), jnp.float32),
    )(hacc, xls, dinv, b1r, w2r)


def _head_call(ap, x2, dv, b2r, wf1, bf1r, wf2, bf2r, n, d2, out):
    nchk = ap.shape[0]
    nw = ap.shape[1]
    ch = ap.shape[2]

    def body(ap_ref, x2_ref, dv_ref, b2_ref, wf1_ref, bf1_ref,
             wf2_ref, bf2_ref, out_ref, z1):
        i = pl.program_id(0)
        asum = jnp.sum(ap_ref[0], axis=0)
        pre = dv_ref[0, 0, :] * (asum + x2_ref[0, 0, :]) + b2_ref[0, 0]
        h2 = _leaky(pre)
        part = jnp.dot(h2[None, :], wf1_ref[...],
                       preferred_element_type=jnp.float32)

        @pl.when(i == 0)
        def _():
            z1[0:1, :] = part

        @pl.when(i > 0)
        def _():
            z1[0:1, :] = z1[0:1, :] + part

        @pl.when(i == pl.num_programs(0) - 1)
        def _():
            zz = z1[0:1, :] + bf1_ref[...]
            aa = _leaky(zz)
            z2 = jnp.dot(aa, wf2_ref[...],
                         preferred_element_type=jnp.float32) + bf2_ref[...]
            a2 = _leaky(z2)
            m = jnp.max(a2, axis=1, keepdims=True)
            ex = jnp.exp(a2 - m)
            out_ref[...] = ex / jnp.sum(ex, axis=1, keepdims=True)

    return pl.pallas_call(
        body,
        grid=(nchk,),
        in_specs=[
            pl.BlockSpec((1, nw, ch), lambda i: (i, 0, 0)),
            pl.BlockSpec((1, 1, ch), lambda i: (i, 0, 0)),
            pl.BlockSpec((1, 1, ch), lambda i: (i, 0, 0)),
            pl.BlockSpec((1, 1), lambda i: (0, 0)),
            pl.BlockSpec((ch, d2), lambda i: (i, 0)),
            pl.BlockSpec((1, d2), lambda i: (0, 0)),
            pl.BlockSpec(wf2.shape, lambda i: (0, 0)),
            pl.BlockSpec((1, out), lambda i: (0, 0)),
        ],
        out_specs=pl.BlockSpec((1, out), lambda i: (0, 0)),
        out_shape=jax.ShapeDtypeStruct((1, out), jnp.float32),
        scratch_shapes=[pltpu.VMEM((8, d2), jnp.float32)],
    )(ap, x2, dv, b2r, wf1, bf1r, wf2, bf2r)


def kernel(x, edge_index, edge_attr, batch, W1, b1, W2, b2, Wf1, bf1, Wf2, bf2):
    n, d = x.shape
    h = W1.shape[1]
    d2 = Wf1.shape[1]
    out = Wf2.shape[1]
    edges = edge_index.shape[1]

    npad = -(-n // (NS * CHUNK)) * (NS * CHUNK)
    nch = -(-edges // (NW * CHUNK))
    nch = nch + (nch % 2)
    epad = NW * nch * CHUNK - edges

    row = edge_index[0]
    col = edge_index[1]
    pidx = jnp.arange(epad, dtype=jnp.int32) % n
    rowp = jnp.concatenate([row, pidx]).reshape(NW, nch, CHUNK)
    colp = jnp.concatenate([col, pidx]).reshape(NW, nch, CHUNK)
    ewp = jnp.concatenate(
        [edge_attr, jnp.zeros((epad,), jnp.float32)]).reshape(NW, nch, CHUNK)

    hpad = 128
    W1p = jnp.pad(W1, ((0, 0), (0, hpad - h)))
    b1p = jnp.pad(b1, (0, hpad - h))
    W2p = jnp.pad(W2.reshape(-1), (0, hpad - h))

    zn = jnp.zeros((npad // NS,), jnp.float32)
    zrow = jnp.zeros((CHUNK, hpad), jnp.float32)
    xp = jnp.pad(x, ((0, npad - n), (0, 0)))

    degp = _deg_call(colp, ewp, zn, npad, nch)
    xls, dinv = _lin1_call(xp, W1p, degp, npad, d, hpad)

    hacc = _prop1_call(rowp, colp, ewp, xls, zrow, npad, nch, hpad, h)
    x2s = _lin2_call(hacc, xls, dinv, b1p.reshape(1, hpad),
                     W2p.reshape(1, hpad), npad, hpad)

    aacc = _prop2_call(rowp, colp, ewp, x2s.reshape(npad), npad, nch)

    ch = 2000
    ng = n // ch
    ap = aacc[:, :n].reshape(NW, ng, ch).transpose(1, 0, 2)
    x2r = x2s[0, :n].reshape(ng, 1, ch)
    dvr = dinv[0, :n].reshape(ng, 1, ch)
    return _head_call(ap, x2r, dvr, b2.reshape(1, 1), Wf1,
                      bf1.reshape(1, d2), Wf2, bf2.reshape(1, out),
                      n, d2, out)


# speedup vs baseline: 33.5731x; 1.0118x over previous
import functools

import jax
import jax.numpy as jnp
from jax import lax
from jax.experimental import pallas as pl
from jax.experimental.pallas import tpu as pltpu
from jax.experimental.pallas import tpu_sc as plsc

NC = 2
NS = 16
LANES = 16
NW = NC * NS
CHUNK = 128


def _leaky(v):
    return jnp.where(v >= 0, v, 0.01 * v)


def _sc_mesh():
    return plsc.VectorSubcoreMesh(core_axis_name="c", subcore_axis_name="s")


def _deg_call(col3, ew3, npad, nch):

    @functools.partial(
        pl.kernel,
        out_type=jax.ShapeDtypeStruct((NW, npad), jnp.float32),
        mesh=_sc_mesh(),
        compiler_params=pltpu.CompilerParams(needs_layout_passes=False),
        scratch_types=[
            pltpu.VMEM((nch, CHUNK), jnp.int32),
            pltpu.VMEM((nch, CHUNK), jnp.float32),
            pltpu.VMEM((npad,), jnp.float32),
        ],
    )
    def deg_k(col3_hbm, ew3_hbm, degp_hbm, colv, ewv, acc):
        cid = lax.axis_index("c")
        sid = lax.axis_index("s")
        wid = sid * NC + cid
        pltpu.sync_copy(col3_hbm.at[wid], colv)
        pltpu.sync_copy(ew3_hbm.at[wid], ewv)

        zv16 = jnp.zeros((LANES,), jnp.float32)

        def zero(i, carry):
            acc[pl.ds(i * LANES, LANES)] = zv16
            return carry

        lax.fori_loop(0, npad // LANES, zero, 0)

        def chunk(j, carry):
            for t in range(CHUNK // LANES):
                sl = pl.ds(t * LANES, LANES)
                plsc.addupdate_scatter(acc, [colv[j, sl]], ewv[j, sl])
            return carry

        lax.fori_loop(0, nch, chunk, 0)
        pltpu.sync_copy(acc, degp_hbm.at[wid])

    return deg_k(col3, ew3)


def _prop1_call(row3, col3, ew3, xls, zrow, npad, nch, h, hreal):
    rpt = npad // NS
    SUB = CHUNK // 2

    @functools.partial(
        pl.kernel,
        out_type=jax.ShapeDtypeStruct((NC, npad, h), jnp.float32),
        mesh=_sc_mesh(),
        scratch_types=[
            pltpu.VMEM((nch, CHUNK), jnp.int32),
            pltpu.VMEM((nch, CHUNK), jnp.int32),
            pltpu.VMEM((nch, CHUNK), jnp.float32),
            pltpu.VMEM((SUB, h), jnp.float32),
            pltpu.VMEM((SUB, h), jnp.float32),
            pltpu.SemaphoreType.DMA,
            pltpu.SemaphoreType.DMA,
            pltpu.VMEM_SHARED((npad, h), jnp.float32),
        ],
    )
    def prop1_k(row3_hbm, col3_hbm, ew3_hbm, xls_hbm, zrow_hbm, hacc_hbm,
                rowv, colv, ewv, buf0, buf1, sem0, sem1, acc):
        cid = lax.axis_index("c")
        sid = lax.axis_index("s")
        wid = sid * NC + cid
        pltpu.sync_copy(row3_hbm.at[wid], rowv)
        pltpu.sync_copy(col3_hbm.at[wid], colv)
        pltpu.sync_copy(ew3_hbm.at[wid], ewv)
        for q in range(rpt // CHUNK):
            pltpu.sync_copy(
                zrow_hbm, acc.at[pl.ds(sid * rpt + q * CHUNK, CHUNK)])
        plsc.subcore_barrier()

        def scale_scatter(j, half, buf):
            def scale(t, c2):
                w16 = ewv[j, pl.ds(half * SUB + t * LANES, LANES)]
                for i in range(LANES):
                    e = t * LANES + i
                    s = w16[i]
                    for q in range(hreal // LANES):
                        sl = pl.ds(q * LANES, LANES)
                        buf[e, sl] = buf[e, sl] * s
                return c2

            lax.fori_loop(0, SUB // LANES, scale, 0)
            pltpu.sync_copy(
                buf, acc.at[colv.at[j, pl.ds(half * SUB, SUB)]], add=True)

        pltpu.async_copy(
            xls_hbm.at[rowv.at[0, pl.ds(0, SUB)]], buf0, sem0)

        def step(j, carry):
            pltpu.make_async_copy(
                xls_hbm.at[rowv.at[j, pl.ds(0, SUB)]], buf0, sem0).wait()
            pltpu.async_copy(
                xls_hbm.at[rowv.at[j, pl.ds(SUB, SUB)]], buf1, sem1)
            scale_scatter(j, 0, buf0)
            pltpu.make_async_copy(
                xls_hbm.at[rowv.at[j, pl.ds(SUB, SUB)]], buf1, sem1).wait()
            jn = jnp.where(j + 1 < nch, j + 1, 0)
            pltpu.async_copy(
                xls_hbm.at[rowv.at[jn, pl.ds(0, SUB)]], buf0, sem0)
            scale_scatter(j, 1, buf1)
            return carry

        lax.fori_loop(0, nch, step, 0)
        pltpu.make_async_copy(
            xls_hbm.at[rowv.at[0, pl.ds(0, SUB)]], buf0, sem0).wait()
        plsc.subcore_barrier()
        for q in range(rpt // SUB):
            off = sid * rpt + q * SUB
            pltpu.sync_copy(acc.at[pl.ds(off, SUB)], buf0)
            pltpu.sync_copy(buf0, hacc_hbm.at[cid, pl.ds(off, SUB)])

    return prop1_k(row3, col3, ew3, xls, zrow)


def _prop2_call(row3, col3, ew3, x2s, npad, nch):

    @functools.partial(
        pl.kernel,
        out_type=jax.ShapeDtypeStruct((NW, npad), jnp.float32),
        mesh=_sc_mesh(),
        compiler_params=pltpu.CompilerParams(needs_layout_passes=False),
        scratch_types=[
            pltpu.VMEM((nch, CHUNK), jnp.int32),
            pltpu.VMEM((nch, CHUNK), jnp.int32),
            pltpu.VMEM((nch, CHUNK), jnp.float32),
            pltpu.VMEM((npad,), jnp.float32),
            pltpu.VMEM((npad,), jnp.float32),
        ],
    )
    def prop2_k(row3_hbm, col3_hbm, ew3_hbm, x2s_hbm, aacc_hbm,
                rowv, colv, ewv, x2v, acc):
        cid = lax.axis_index("c")
        sid = lax.axis_index("s")
        wid = sid * NC + cid
        pltpu.sync_copy(row3_hbm.at[wid], rowv)
        pltpu.sync_copy(col3_hbm.at[wid], colv)
        pltpu.sync_copy(ew3_hbm.at[wid], ewv)
        pltpu.sync_copy(x2s_hbm, x2v)

        zv16 = jnp.zeros((LANES,), jnp.float32)

        def zero(i, carry):
            acc[pl.ds(i * LANES, LANES)] = zv16
            return carry

        lax.fori_loop(0, npad // LANES, zero, 0)

        def chunk(j, carry):
            for t in range(CHUNK // LANES):
                sl = pl.ds(t * LANES, LANES)
                vals = plsc.load_gather(x2v, [rowv[j, sl]])
                plsc.addupdate_scatter(
                    acc, [colv[j, sl]], vals * ewv[j, sl])
            return carry

        lax.fori_loop(0, nch, chunk, 0)
        pltpu.sync_copy(acc, aacc_hbm.at[wid])

    return prop2_k(row3, col3, ew3, x2s)


def _lin1_call(xp, w1, degp, npad, d, h):
    blk = 256
    grid = (npad // blk,)

    def body(x_ref, w1_ref, degp_ref, xls_ref, dinv_ref):
        deg = jnp.sum(degp_ref[...], axis=0) + 1.0
        dinv = jnp.where(deg > 0, lax.rsqrt(deg), 0.0)
        xl = jnp.dot(x_ref[...], w1_ref[...], preferred_element_type=jnp.float32)
        xls_ref[...] = xl * dinv[:, None]
        dinv_ref[...] = dinv[None, :]

    return pl.pallas_call(
        body,
        grid=grid,
        in_specs=[
            pl.BlockSpec((blk, d), lambda i: (i, 0)),
            pl.BlockSpec((d, h), lambda i: (0, 0)),
            pl.BlockSpec((NW, blk), lambda i: (0, i)),
        ],
        out_specs=[
            pl.BlockSpec((blk, h), lambda i: (i, 0)),
            pl.BlockSpec((1, blk), lambda i: (0, i)),
        ],
        out_shape=[
            jax.ShapeDtypeStruct((npad, h), jnp.float32),
            jax.ShapeDtypeStruct((1, npad), jnp.float32),
        ],
    )(xp, w1, degp)


def _lin2_call(hacc, xls, dinv, b1r, w2r, npad, h):
    blk = 256
    grid = (npad // blk,)

    def body(accp_ref, xls_ref, dinv_ref, b1_ref, w2_ref, x2s_ref):
        ea = accp_ref[0] + accp_ref[1]
        dinv = dinv_ref[0, :]
        pre = dinv[:, None] * (ea + xls_ref[...]) + b1_ref[0, :][None, :]
        h1 = _leaky(pre)
        xl2 = jnp.sum(h1 * w2_ref[0, :][None, :], axis=1)
        x2s_ref[...] = (dinv * xl2)[None, :]

    return pl.pallas_call(
        body,
        grid=grid,
        in_specs=[
            pl.BlockSpec((NC, blk, h), lambda i: (0, i, 0)),
            pl.BlockSpec((blk, h), lambda i: (i, 0)),
            pl.BlockSpec((1, blk), lambda i: (0, i)),
            pl.BlockSpec((1, h), lambda i: (0, 0)),
            pl.BlockSpec((1, h), lambda i: (0, 0)),
        ],
        out_specs=pl.BlockSpec((1, blk), lambda i: (0, i)),
        out_shape=jax.ShapeDtypeStruct((1, npad), jnp.float32),
    )(hacc, xls, dinv, b1r, w2r)


def _head_call(ap, x2, dv, b2r, wf1, bf1r, wf2, bf2r, n, d2, out):
    nchk = ap.shape[0]
    nw = ap.shape[1]
    ch = ap.shape[2]

    def body(ap_ref, x2_ref, dv_ref, b2_ref, wf1_ref, bf1_ref,
             wf2_ref, bf2_ref, out_ref, z1):
        i = pl.program_id(0)
        asum = jnp.sum(ap_ref[0], axis=0)
        pre = dv_ref[0, 0, :] * (asum + x2_ref[0, 0, :]) + b2_ref[0, 0]
        h2 = _leaky(pre)
        part = jnp.dot(h2[None, :], wf1_ref[...],
                       preferred_element_type=jnp.float32)

        @pl.when(i == 0)
        def _():
            z1[0:1, :] = part

        @pl.when(i > 0)
        def _():
            z1[0:1, :] = z1[0:1, :] + part

        @pl.when(i == pl.num_programs(0) - 1)
        def _():
            zz = z1[0:1, :] + bf1_ref[...]
            aa = _leaky(zz)
            z2 = jnp.dot(aa, wf2_ref[...],
                         preferred_element_type=jnp.float32) + bf2_ref[...]
            a2 = _leaky(z2)
            m = jnp.max(a2, axis=1, keepdims=True)
            ex = jnp.exp(a2 - m)
            out_ref[...] = ex / jnp.sum(ex, axis=1, keepdims=True)

    return pl.pallas_call(
        body,
        grid=(nchk,),
        in_specs=[
            pl.BlockSpec((1, nw, ch), lambda i: (i, 0, 0)),
            pl.BlockSpec((1, 1, ch), lambda i: (i, 0, 0)),
            pl.BlockSpec((1, 1, ch), lambda i: (i, 0, 0)),
            pl.BlockSpec((1, 1), lambda i: (0, 0)),
            pl.BlockSpec((ch, d2), lambda i: (i, 0)),
            pl.BlockSpec((1, d2), lambda i: (0, 0)),
            pl.BlockSpec(wf2.shape, lambda i: (0, 0)),
            pl.BlockSpec((1, out), lambda i: (0, 0)),
        ],
        out_specs=pl.BlockSpec((1, out), lambda i: (0, 0)),
        out_shape=jax.ShapeDtypeStruct((1, out), jnp.float32),
        scratch_shapes=[pltpu.VMEM((8, d2), jnp.float32)],
    )(ap, x2, dv, b2r, wf1, bf1r, wf2, bf2r)


def kernel(x, edge_index, edge_attr, batch, W1, b1, W2, b2, Wf1, bf1, Wf2, bf2):
    n, d = x.shape
    h = W1.shape[1]
    d2 = Wf1.shape[1]
    out = Wf2.shape[1]
    edges = edge_index.shape[1]

    npad = -(-n // (NS * CHUNK)) * (NS * CHUNK)
    nch = -(-edges // (NW * CHUNK))
    nch = nch + (nch % 2)
    epad = NW * nch * CHUNK - edges

    row = edge_index[0]
    col = edge_index[1]
    pidx = jnp.arange(epad, dtype=jnp.int32) % n
    rowp = jnp.concatenate([row, pidx]).reshape(NW, nch, CHUNK)
    colp = jnp.concatenate([col, pidx]).reshape(NW, nch, CHUNK)
    ewp = jnp.concatenate(
        [edge_attr, jnp.zeros((epad,), jnp.float32)]).reshape(NW, nch, CHUNK)

    hpad = 128
    W1p = jnp.pad(W1, ((0, 0), (0, hpad - h)))
    b1p = jnp.pad(b1, (0, hpad - h))
    W2p = jnp.pad(W2.reshape(-1), (0, hpad - h))

    zrow = jnp.zeros((CHUNK, hpad), jnp.float32)
    xp = jnp.pad(x, ((0, npad - n), (0, 0)))

    degp = _deg_call(colp, ewp, npad, nch)
    xls, dinv = _lin1_call(xp, W1p, degp, npad, d, hpad)

    hacc = _prop1_call(rowp, colp, ewp, xls, zrow, npad, nch, hpad, h)
    x2s = _lin2_call(hacc, xls, dinv, b1p.reshape(1, hpad),
                     W2p.reshape(1, hpad), npad, hpad)

    aacc = _prop2_call(rowp, colp, ewp, x2s.reshape(npad), npad, nch)

    ch = 2000
    ng = n // ch
    ap = aacc[:, :n].reshape(NW, ng, ch).transpose(1, 0, 2)
    x2r = x2s[0, :n].reshape(ng, 1, ch)
    dvr = dinv[0, :n].reshape(ng, 1, ch)
    return _head_call(ap, x2r, dvr, b2.reshape(1, 1), Wf1,
                      bf1.reshape(1, d2), Wf2, bf2.reshape(1, out),
                      n, d2, out)
